# baseline (speedup 1.0000x reference)
import jax
import jax.numpy as jnp
import numpy as np
from jax import lax
from jax.experimental import pallas as pl
from jax.experimental.pallas import tpu as pltpu
from jax.experimental.pallas import tpu_sc as plsc

MAX_JETS = 16
B = 16384
D = 16
P = MAX_JETS * (MAX_JETS - 1) // 2

NC = 2
NS = 16
NW = NC * NS
E = B // NW
CH = 128
NCH = E // CH
PC = 20
NPC = P // PC


def _build_pk_table() -> np.ndarray:
    tab = np.full((MAX_JETS - 1, 128), (16 << 6) | (16 << 1), dtype=np.int32)
    for n in range(2, MAX_JETS + 1):
        p = 0
        for j in range(n):
            for k in range(j + 1, n):
                tab[n - 2, p] = (j << 6) | (k << 1)
                p += 1
    return tab


_PK_TAB = _build_pk_table().reshape(-1)


def _sc_body(x_hbm, n_hbm, pk_hbm, out_hbm, pn_hbm,
             pk_v, n_v, pn_v, x_v, o_v0, o_v1, sout0, sout1):
    cid = lax.axis_index("c")
    sid = lax.axis_index("s")
    wid = sid * NC + cid
    ev0 = wid * E

    pltpu.sync_copy(pk_hbm, pk_v)
    pltpu.sync_copy(n_hbm.at[pl.ds(ev0, E)], n_v)

    def pn_body(c, carry):
        nv = n_v[pl.ds(c * 16, 16)]
        ones = jnp.full((16,), 1, dtype=jnp.int32)
        pn = (nv * (nv - ones)) >> ones
        pn_v[pl.ds(c * 16, 16)] = pn.astype(jnp.float32)
        return carry
    lax.fori_loop(0, E // 16, pn_body, None)
    pltpu.sync_copy(pn_v, pn_hbm.at[pl.ds(ev0, E)])

    lanes = lax.iota(jnp.int32, 16)
    two = jnp.full((16,), 2, jnp.int32)
    c128 = jnp.full((16,), 128, jnp.int32)
    six = jnp.full((16,), 6, jnp.int32)
    one = jnp.full((16,), 1, jnp.int32)
    c31 = jnp.full((16,), 31, jnp.int32)
    zvec = jnp.zeros((16,), jnp.float32)

    for d in range(D):
        for l in range(CH // 16):
            x_v[MAX_JETS, d, pl.ds(l * 16, 16)] = zvec

    obufs = (o_v0, o_v1)
    souts = (sout0, sout1)
    dfs = [jnp.full((16,), d, jnp.int32) for d in range(D)]

    def compute_tile(c, p0, o_v):
        def lane_body(l, carry3):
            boff = l * 16 + lanes
            nm2 = n_v[pl.ds(c * CH + l * 16, 16)] - two
            tb = nm2 * c128
            pks = [plsc.load_gather(pk_v, [tb + (p0 + dp)])
                   for dp in range(PC)]
            for dp in range(PC):
                pk = pks[dp]
                jt = pk >> six
                kt = (pk >> one) & c31
                del jt, kt
                for dh in range(D // 8):
                    dr = range(dh * 8, dh * 8 + 8)
                    gjs = [x_v[3, d, pl.ds(l * 16, 16)] for d in dr]
                    gks = [x_v[7, d, pl.ds(l * 16, 16)] for d in dr]
                    for i, d in enumerate(dr):
                        o_v[dp, d, pl.ds(l * 16, 16)] = gjs[i] + gks[i]
            return carry3
        lax.fori_loop(0, CH // 16, lane_body, None)

    def chunk_body(c, carry):
        b0 = ev0 + c * CH
        pltpu.sync_copy(x_hbm.at[:, :, pl.ds(b0, CH)],
                        x_v.at[pl.ds(0, MAX_JETS)])

        def ptile_body(pcc, carry2):
            for par in range(2):
                pc = pcc * 2 + par
                p0 = pc * PC
                o_v, sout = obufs[par], souts[par]
                t = c * NPC + pc

                del sout, t
                compute_tile(c, p0, o_v)
            return carry2
        lax.fori_loop(0, NPC // 2, ptile_body, None)
        return carry
    lax.fori_loop(0, NCH, chunk_body, None)

    pltpu.sync_copy(o_v0, out_hbm.at[pl.ds(0, PC), :, pl.ds(ev0, CH)])


@jax.jit
def _run(x_t, n_i32, pk):
    mesh = plsc.VectorSubcoreMesh(core_axis_name="c", subcore_axis_name="s")
    out_t, pn = pl.kernel(
        _sc_body,
        out_type=[
            jax.ShapeDtypeStruct((P, D, B), jnp.float32),
            jax.ShapeDtypeStruct((B,), jnp.float32),
        ],
        mesh=mesh,
        compiler_params=pltpu.CompilerParams(needs_layout_passes=False),
        scratch_types=[
            pltpu.VMEM((15 * 128,), jnp.int32),
            pltpu.VMEM((E,), jnp.int32),
            pltpu.VMEM((E,), jnp.float32),
            pltpu.VMEM((MAX_JETS + 1, D, CH), jnp.float32),
            pltpu.VMEM((PC, D, CH), jnp.float32),
            pltpu.VMEM((PC, D, CH), jnp.float32),
            pltpu.SemaphoreType.DMA,
            pltpu.SemaphoreType.DMA,
        ],
    )(x_t, n_i32, pk)
    return out_t, pn


def kernel(inputs, dict_vals, jet_num):
    del dict_vals
    x_t = jnp.transpose(inputs, (1, 2, 0))
    n_i32 = jet_num.astype(jnp.int32)
    pk = jnp.asarray(_PK_TAB)
    out_t, pn = _run(x_t, n_i32, pk)
    pairs_sum = jnp.transpose(out_t, (2, 0, 1))
    return pairs_sum, pn.reshape(B, 1)


# speedup vs baseline: 1.1254x; 1.1254x over previous
import jax
import jax.numpy as jnp
import numpy as np
from jax import lax
from jax.experimental import pallas as pl
from jax.experimental.pallas import tpu as pltpu
from jax.experimental.pallas import tpu_sc as plsc

MAX_JETS = 16
B = 16384
D = 16
P = MAX_JETS * (MAX_JETS - 1) // 2

NC = 2
NS = 16
NW = NC * NS
E = B // NW
CH = 128
NCH = E // CH
PC = 20
NPC = P // PC


def _build_pk_table() -> np.ndarray:
    tab = np.full((MAX_JETS - 1, 128), (16 << 6) | (16 << 1), dtype=np.int32)
    for n in range(2, MAX_JETS + 1):
        p = 0
        for j in range(n):
            for k in range(j + 1, n):
                tab[n - 2, p] = (j << 6) | (k << 1)
                p += 1
    return tab


_PK_TAB = _build_pk_table().reshape(-1)


def _sc_body(x_hbm, n_hbm, pk_hbm, out_hbm, pn_hbm,
             pk_v, n_v, pn_v, x_v, o_v0, o_v1, sout0, sout1):
    cid = lax.axis_index("c")
    sid = lax.axis_index("s")
    wid = sid * NC + cid
    ev0 = wid * E

    pltpu.sync_copy(pk_hbm, pk_v)
    pltpu.sync_copy(n_hbm.at[pl.ds(ev0, E)], n_v)

    def pn_body(c, carry):
        nv = n_v[pl.ds(c * 16, 16)]
        ones = jnp.full((16,), 1, dtype=jnp.int32)
        pn = (nv * (nv - ones)) >> ones
        pn_v[pl.ds(c * 16, 16)] = pn.astype(jnp.float32)
        return carry
    lax.fori_loop(0, E // 16, pn_body, None)
    pltpu.sync_copy(pn_v, pn_hbm.at[pl.ds(ev0, E)])

    lanes = lax.iota(jnp.int32, 16)
    two = jnp.full((16,), 2, jnp.int32)
    c128 = jnp.full((16,), 128, jnp.int32)
    six = jnp.full((16,), 6, jnp.int32)
    one = jnp.full((16,), 1, jnp.int32)
    c31 = jnp.full((16,), 31, jnp.int32)
    zvec = jnp.zeros((16,), jnp.float32)

    for d in range(D):
        for l in range(CH // 16):
            x_v[MAX_JETS, d, pl.ds(l * 16, 16)] = zvec

    obufs = (o_v0, o_v1)
    souts = (sout0, sout1)
    dfs = [jnp.full((16,), d, jnp.int32) for d in range(D)]

    def compute_tile(c, p0, o_v):
        def lane_body(l, carry3):
            boff = l * 16 + lanes
            nm2 = n_v[pl.ds(c * CH + l * 16, 16)] - two
            tb = nm2 * c128
            pks = [plsc.load_gather(pk_v, [tb + (p0 + dp)])
                   for dp in range(PC)]
            for dp in range(PC):
                pk = pks[dp]
                jt = pk >> six
                kt = (pk >> one) & c31
                for dh in range(D // 8):
                    dr = range(dh * 8, dh * 8 + 8)
                    gjs = [plsc.load_gather(x_v, [jt, dfs[d], boff])
                           for d in dr]
                    gks = [plsc.load_gather(x_v, [kt, dfs[d], boff])
                           for d in dr]
                    for i, d in enumerate(dr):
                        o_v[dp, d, pl.ds(l * 16, 16)] = gjs[i] + gks[i]
            return carry3
        lax.fori_loop(0, CH // 16, lane_body, None)

    def chunk_body(c, carry):
        b0 = ev0 + c * CH
        pltpu.sync_copy(x_hbm.at[:, :, pl.ds(b0, CH)],
                        x_v.at[pl.ds(0, MAX_JETS)])

        def ptile_body(pcc, carry2):
            for par in range(2):
                pc = pcc * 2 + par
                p0 = pc * PC
                o_v, sout = obufs[par], souts[par]
                t = c * NPC + pc

                @pl.when(t >= 2)
                def _drain():
                    wrap = pc < 2
                    pb0 = ev0 + jnp.where(wrap, c - 1, c) * CH
                    pp0 = jnp.where(wrap, pc - 2 + NPC, pc - 2) * PC
                    pltpu.make_async_copy(
                        o_v, out_hbm.at[pl.ds(pp0, PC), :, pl.ds(pb0, CH)],
                        sout).wait()

                compute_tile(c, p0, o_v)
                pltpu.async_copy(
                    o_v, out_hbm.at[pl.ds(p0, PC), :, pl.ds(b0, CH)], sout)
            return carry2
        lax.fori_loop(0, NPC // 2, ptile_body, None)
        return carry
    lax.fori_loop(0, NCH, chunk_body, None)

    tlast = NCH * NPC
    for par in range(2):
        tp = tlast - 2 + par
        pb0 = ev0 + (tp // NPC) * CH
        pp0 = (tp % NPC) * PC
        pltpu.make_async_copy(
            obufs[par], out_hbm.at[pl.ds(pp0, PC), :, pl.ds(pb0, CH)],
            souts[par]).wait()


@jax.jit
def _run(x_t, n_i32, pk):
    mesh = plsc.VectorSubcoreMesh(core_axis_name="c", subcore_axis_name="s")
    out_t, pn = pl.kernel(
        _sc_body,
        out_type=[
            jax.ShapeDtypeStruct((P, D, B), jnp.float32),
            jax.ShapeDtypeStruct((B,), jnp.float32),
        ],
        mesh=mesh,
        compiler_params=pltpu.CompilerParams(needs_layout_passes=False),
        scratch_types=[
            pltpu.VMEM((15 * 128,), jnp.int32),
            pltpu.VMEM((E,), jnp.int32),
            pltpu.VMEM((E,), jnp.float32),
            pltpu.VMEM((MAX_JETS + 1, D, CH), jnp.float32),
            pltpu.VMEM((PC, D, CH), jnp.float32),
            pltpu.VMEM((PC, D, CH), jnp.float32),
            pltpu.SemaphoreType.DMA,
            pltpu.SemaphoreType.DMA,
        ],
    )(x_t, n_i32, pk)
    return out_t, pn


def kernel(inputs, dict_vals, jet_num):
    del dict_vals
    x_t = jnp.transpose(inputs, (1, 2, 0))
    n_i32 = jet_num.astype(jnp.int32)
    pk = jnp.asarray(_PK_TAB)
    out_t, pn = _run(x_t, n_i32, pk)
    pairs_sum = jnp.transpose(out_t, (2, 0, 1))
    return pairs_sum, pn.reshape(B, 1)
